# Initial kernel scaffold; baseline (speedup 1.0000x reference)
import jax
import jax.numpy as jnp
from jax.experimental import pallas as pl


def kernel(x_input, x_function, x_output, edge_index_if, edge_index_fi, edge_index_ff, edge_index_of, edge_index_fo, batch, mask, Wl0, bl0, Wr0, Wl, bl, Wr, ln_w, ln_b, att_w, lin_w, lin_b):
    raise NotImplementedError("write your pallas kernel here")


# speedup vs baseline: 5.1954x; 5.1954x over previous
import functools

import jax
import jax.numpy as jnp
from jax import lax
from jax.experimental import pallas as pl
from jax.experimental.pallas import tpu as pltpu
from jax.experimental.pallas import tpu_sc as plsc

N_IN = 10000
N_FN = 50000
N_OUT = 10000
D = 128
H = 32
NH = 10
NG = 32

NP_F = 50176
NP_I = 10240
NP_O = 10240
BF = 3136
BFU = 1568
BI = 1280

CH = 512
ZR = 256
NC = 2
NS = 16
NW = NC * NS

_HI = jax.lax.Precision.HIGHEST


def _dot(a, b):
    return jnp.dot(a, b, precision=_HI)


def _dotd(a, b):
    return jnp.dot(a, b, precision=_HI)


def _erf(x):
    ax = jnp.abs(x)
    t = 1.0 / (1.0 + 0.3275911 * ax)
    poly = t * (0.254829592 + t * (-0.284496736 + t * (1.421413741
                + t * (-1.453152027 + t * 1.061405429))))
    y = 1.0 - poly * jnp.exp(-ax * ax)
    return jnp.sign(x) * y


def _gelu(x):
    return 0.5 * x * (1.0 + _erf(x * 0.7071067811865476))


def _ln(x, w, b):
    mu = jnp.mean(x, axis=-1, keepdims=True)
    xc = x - mu
    var = jnp.mean(xc * xc, axis=-1, keepdims=True)
    return xc / jnp.sqrt(var + 1e-5) * w + b


_REL_CHUNKS = (10, 30, 10, 10, 10)
_REL_NPAD = (NP_F, NP_F, NP_F, NP_I, NP_O)


def _sc_mesh():
    return plsc.VectorSubcoreMesh(core_axis_name="c", subcore_axis_name="s")


def _zero_slice(acc, zbuf, base, zr):
    nfull, rem = zr // ZR, zr % ZR
    for kk in range(nfull):
        pltpu.sync_copy(zbuf, acc.at[pl.ds(base + kk * ZR, ZR)])
    if rem:
        pltpu.sync_copy(zbuf.at[pl.ds(0, rem)], acc.at[pl.ds(base + nfull * ZR, rem)])


def _sc_agg_call(tables, idx_pairs, zeros_hbm):
    out_types = [jax.ShapeDtypeStruct((NC, npad, H), jnp.float32) for npad in _REL_NPAD]
    scratch = [
        pltpu.VMEM((CH,), jnp.int32),
        pltpu.VMEM((CH,), jnp.int32),
        pltpu.VMEM((CH, H), jnp.float32),
        pltpu.VMEM((ZR, H), jnp.float32),
        pltpu.VMEM_SHARED((NP_F, H), jnp.float32),
        pltpu.SemaphoreType.DMA,
    ]

    @functools.partial(pl.kernel, out_type=out_types, mesh=_sc_mesh(),
                       scratch_types=scratch,
                       compiler_params=pltpu.CompilerParams(
                           use_tc_tiling_on_sc=False))
    def k(*refs):
        tbls = refs[0:5]
        idxs = refs[5:15]
        zz = refs[15]
        outs = refs[16:21]
        sidx, didx, rows, zbuf, acc, sem = refs[21:27]
        c = lax.axis_index("c")
        s = lax.axis_index("s")
        w = c * NS + s
        pltpu.sync_copy(zz, zbuf)
        for r in range(5):
            tbl, sh, dh, oh = tbls[r], idxs[2 * r], idxs[2 * r + 1], outs[r]
            nchunks, npad = _REL_CHUNKS[r], _REL_NPAD[r]
            zr = npad // NS
            base = s * zr
            _zero_slice(acc, zbuf, base, zr)
            plsc.subcore_barrier()
            perw = nchunks * CH
            for kk in range(nchunks):
                off = w * perw + kk * CH
                pltpu.sync_copy(sh.at[pl.ds(off, CH)], sidx)
                pltpu.sync_copy(dh.at[pl.ds(off, CH)], didx)
                pltpu.async_copy(tbl.at[sidx], rows, sem).wait()
                pltpu.sync_copy(rows, acc.at[didx], add=True)
            plsc.subcore_barrier()
            pltpu.sync_copy(acc.at[pl.ds(base, zr)], oh.at[c, pl.ds(base, zr)])
            plsc.subcore_barrier()

    args = list(tables)
    for sidx, didx in idx_pairs:
        args += [sidx, didx]
    args.append(zeros_hbm)
    return k(*args)


def _sc_count_call(idx_dsts, ones_hbm, zeros_hbm):
    out_types = [jax.ShapeDtypeStruct((NC, npad, H), jnp.float32) for npad in _REL_NPAD]
    scratch = [
        pltpu.VMEM((CH,), jnp.int32),
        pltpu.VMEM((CH, H), jnp.float32),
        pltpu.VMEM((ZR, H), jnp.float32),
        pltpu.VMEM_SHARED((NP_F, H), jnp.float32),
        pltpu.SemaphoreType.DMA,
    ]

    @functools.partial(pl.kernel, out_type=out_types, mesh=_sc_mesh(),
                       scratch_types=scratch,
                       compiler_params=pltpu.CompilerParams(
                           use_tc_tiling_on_sc=False))
    def k(*refs):
        idxs = refs[0:5]
        oo = refs[5]
        zz = refs[6]
        outs = refs[7:12]
        didx, rows, zbuf, acc, sem = refs[12:17]
        c = lax.axis_index("c")
        s = lax.axis_index("s")
        w = c * NS + s
        pltpu.sync_copy(zz, zbuf)
        pltpu.sync_copy(oo, rows)
        for r in range(5):
            dh, oh = idxs[r], outs[r]
            nchunks, npad = _REL_CHUNKS[r], _REL_NPAD[r]
            zr = npad // NS
            base = s * zr
            _zero_slice(acc, zbuf, base, zr)
            plsc.subcore_barrier()
            perw = nchunks * CH
            for kk in range(nchunks):
                off = w * perw + kk * CH
                pltpu.sync_copy(dh.at[pl.ds(off, CH)], didx)
                pltpu.sync_copy(rows, acc.at[didx], add=True)
            plsc.subcore_barrier()
            pltpu.sync_copy(acc.at[pl.ds(base, zr)], oh.at[c, pl.ds(base, zr)])
            plsc.subcore_barrier()
    return k(*idx_dsts, ones_hbm, zeros_hbm)


def _init_f_call(xf_pad, m_pad, w3):
    nblk = NP_F // BF

    def body(x_ref, m_ref, w_ref, xfm_ref, t0_ref, t1_ref, t2_ref):
        xm = x_ref[...] * m_ref[...]
        xfm_ref[...] = xm
        t0_ref[...] = _dot(xm, w_ref[0])
        t1_ref[...] = _dot(xm, w_ref[1])
        t2_ref[...] = _dot(xm, w_ref[2])

    return pl.pallas_call(
        body,
        grid=(nblk,),
        in_specs=[
            pl.BlockSpec((BF, D), lambda i: (i, 0)),
            pl.BlockSpec((BF, 1), lambda i: (i, 0)),
            pl.BlockSpec((3, D, H), lambda i: (0, 0, 0)),
        ],
        out_specs=[
            pl.BlockSpec((BF, D), lambda i: (i, 0)),
            pl.BlockSpec((BF, H), lambda i: (i, 0)),
            pl.BlockSpec((BF, H), lambda i: (i, 0)),
            pl.BlockSpec((BF, H), lambda i: (i, 0)),
        ],
        out_shape=[
            jax.ShapeDtypeStruct((NP_F, D), jnp.float32),
            jax.ShapeDtypeStruct((NP_F, H), jnp.float32),
            jax.ShapeDtypeStruct((NP_F, H), jnp.float32),
            jax.ShapeDtypeStruct((NP_F, H), jnp.float32),
        ],
    )(xf_pad, m_pad, w3)


def _init_io_call(x_pad, w):
    npad = x_pad.shape[0]
    nblk = npad // BI

    def body(x_ref, w_ref, t_ref):
        t_ref[...] = _dot(x_ref[...], w_ref[...])

    return pl.pallas_call(
        body,
        grid=(nblk,),
        in_specs=[
            pl.BlockSpec((BI, D), lambda i: (i, 0)),
            pl.BlockSpec((D, H), lambda i: (0, 0)),
        ],
        out_specs=pl.BlockSpec((BI, H), lambda i: (i, 0)),
        out_shape=jax.ShapeDtypeStruct((npad, H), jnp.float32),
    )(x_pad, w)


def _update_f_call(p3, c3, x_cur, m_pad, wl3, wr3, b_sum, ln_w2, ln_b2):
    din = x_cur.shape[1]
    nblk = NP_F // BFU
    apply_wl = wl3 is not None

    def body(*refs):
        (pif, pff, pof, cif, cff, cof, x_ref, m_ref) = refs[:8]
        idx = 8
        if apply_wl:
            wl_ref = refs[idx]
            idx += 1
        wr_ref, b_ref, lw_ref, lb_ref, xout = refs[idx:idx + 5]
        m_if = (pif[0] + pif[1]) / jnp.maximum(cif[0] + cif[1], 1.0)
        m_ff = (pff[0] + pff[1]) / jnp.maximum(cff[0] + cff[1], 1.0)
        m_of = (pof[0] + pof[1]) / jnp.maximum(cof[0] + cof[1], 1.0)
        if apply_wl:
            acc = _dotd(m_if, wl_ref[0]) + _dotd(m_ff, wl_ref[1]) \
                + _dotd(m_of, wl_ref[2])
        else:
            acc = m_if + m_ff + m_of
        x = x_ref[...]
        root = _dotd(x, wr_ref[0]) + _dotd(x, wr_ref[1]) + _dotd(x, wr_ref[2])
        nf = (acc + b_ref[...] + root) * m_ref[...]
        xout[...] = _ln(_gelu(nf), lw_ref[...], lb_ref[...])

    pspec = pl.BlockSpec((NC, BFU, H), lambda i: (0, i, 0))
    in_specs = [pspec] * 6 + [
        pl.BlockSpec((BFU, din), lambda i: (i, 0)),
        pl.BlockSpec((BFU, 1), lambda i: (i, 0)),
    ]
    args = list(p3) + list(c3) + [x_cur, m_pad]
    if apply_wl:
        in_specs.append(pl.BlockSpec((3, H, H), lambda i: (0, 0, 0)))
        args.append(wl3)
    in_specs += [
        pl.BlockSpec((3, din, H), lambda i: (0, 0, 0)),
        pl.BlockSpec((1, H), lambda i: (0, 0)),
        pl.BlockSpec((1, H), lambda i: (0, 0)),
        pl.BlockSpec((1, H), lambda i: (0, 0)),
    ]
    args += [wr3, b_sum, ln_w2, ln_b2]

    return pl.pallas_call(
        body, grid=(nblk,), in_specs=in_specs,
        out_specs=pl.BlockSpec((BFU, H), lambda i: (i, 0)),
        out_shape=jax.ShapeDtypeStruct((NP_F, H), jnp.float32),
    )(*args)


def _update_io_call(p, cnt, x_cur, wl, wr, b, ln_w2, ln_b2):
    din = x_cur.shape[1]
    npad = x_cur.shape[0]
    nblk = npad // BI
    apply_wl = wl is not None

    def body(*refs):
        p_ref, c_ref, x_ref = refs[:3]
        idx = 3
        if apply_wl:
            wl_ref = refs[idx]
            idx += 1
        wr_ref, b_ref, lw_ref, lb_ref, xout = refs[idx:idx + 5]
        mean = (p_ref[0] + p_ref[1]) / jnp.maximum(c_ref[0] + c_ref[1], 1.0)
        acc = _dotd(mean, wl_ref[...]) if apply_wl else mean
        nf = acc + b_ref[...] + _dotd(x_ref[...], wr_ref[...])
        xout[...] = _ln(_gelu(nf), lw_ref[...], lb_ref[...])

    pspec = pl.BlockSpec((NC, BI, H), lambda i: (0, i, 0))
    in_specs = [pspec, pspec,
                pl.BlockSpec((BI, din), lambda i: (i, 0))]
    args = [p, cnt, x_cur]
    if apply_wl:
        in_specs.append(pl.BlockSpec((H, H), lambda i: (0, 0)))
        args.append(wl)
    in_specs += [pl.BlockSpec((din, H), lambda i: (0, 0)),
                 pl.BlockSpec((1, H), lambda i: (0, 0)),
                 pl.BlockSpec((1, H), lambda i: (0, 0)),
                 pl.BlockSpec((1, H), lambda i: (0, 0))]
    args += [wr, b, ln_w2, ln_b2]

    return pl.pallas_call(
        body, grid=(nblk,), in_specs=in_specs,
        out_specs=pl.BlockSpec((BI, H), lambda i: (i, 0)),
        out_shape=jax.ShapeDtypeStruct((npad, H), jnp.float32),
    )(*args)


def _pool_call(xf5, m_pad, batch_row, att_rep, lin_w, lin_b):
    nblk = NP_F // BF
    HW = NH * H

    def body(x_ref, m_ref, b_ref, aw_ref, lw_ref, lb_ref, o_ref, num, den):
        i = pl.program_id(0)

        @pl.when(i == 0)
        def _():
            num[...] = jnp.zeros_like(num)
            den[...] = jnp.zeros_like(den)

        xm = x_ref[...] * m_ref[...]
        s320 = _dotd(xm, aw_ref[...])
        ex = jnp.exp(s320)
        xtile = jnp.concatenate([xm] * NH, axis=1)
        onehot = (b_ref[0] == lax.broadcasted_iota(jnp.int32, (NG, 1), 0))
        onehot = onehot.astype(jnp.float32)
        num[...] += _dot(onehot, ex * xtile)
        den[...] += _dot(onehot, ex)

        @pl.when(i == nblk - 1)
        def _():
            xpool = num[...] / jnp.maximum(den[...], 1e-9)
            o_ref[...] = _dotd(_gelu(xpool), lw_ref[...]) + lb_ref[...]

    return pl.pallas_call(
        body,
        grid=(nblk,),
        in_specs=[
            pl.BlockSpec((BF, H), lambda i: (i, 0)),
            pl.BlockSpec((BF, 1), lambda i: (i, 0)),
            pl.BlockSpec((1, 1, BF), lambda i: (i, 0, 0)),
            pl.BlockSpec((H, HW), lambda i: (0, 0)),
            pl.BlockSpec((HW, 1), lambda i: (0, 0)),
            pl.BlockSpec((1, 1), lambda i: (0, 0)),
        ],
        out_specs=pl.BlockSpec((NG, 1), lambda i: (0, 0)),
        out_shape=jax.ShapeDtypeStruct((NG, 1), jnp.float32),
        scratch_shapes=[
            pltpu.VMEM((NG, HW), jnp.float32),
            pltpu.VMEM((NG, HW), jnp.float32),
        ],
    )(xf5, m_pad, batch_row, att_rep, lin_w, lin_b)


def _pad_rows(x, npad):
    return jnp.pad(x, ((0, npad - x.shape[0]),) + ((0, 0),) * (x.ndim - 1))


def _prep_edges(ei, nchunks, dst_pad):
    tot = NW * nchunks * CH
    e = ei.shape[1]
    src = jnp.pad(ei[0].astype(jnp.int32), (0, tot - e))
    dst = jnp.pad(ei[1].astype(jnp.int32), (0, tot - e),
                  constant_values=dst_pad)
    return src, dst


def kernel(x_input, x_function, x_output, edge_index_if, edge_index_fi,
           edge_index_ff, edge_index_of, edge_index_fo, batch, mask,
           Wl0, bl0, Wr0, Wl, bl, Wr, ln_w, ln_b, att_w, lin_w, lin_b):
    f32 = jnp.float32

    xi = _pad_rows(x_input.astype(f32), NP_I)
    xf = _pad_rows(x_function.astype(f32), NP_F)
    xo = _pad_rows(x_output.astype(f32), NP_O)
    m_pad = _pad_rows(mask.astype(f32)[:, None], NP_F)
    batch_row = jnp.pad(batch.astype(jnp.int32), (0, NP_F - N_FN),
                        constant_values=NG).reshape(NP_F // BF, 1, BF)

    s_if, d_if = _prep_edges(edge_index_if, 10, N_FN)
    s_ff, d_ff = _prep_edges(edge_index_ff, 30, N_FN)
    s_of, d_of = _prep_edges(edge_index_of, 10, N_FN)
    s_fi, d_fi = _prep_edges(edge_index_fi, 10, N_IN)
    s_fo, d_fo = _prep_edges(edge_index_fo, 10, N_OUT)

    zeros_hbm = jnp.zeros((ZR, H), f32)
    ones_hbm = jnp.ones((CH, H), f32)

    ln_w2 = ln_w[None, :].astype(f32)
    ln_b2 = ln_b[None, :].astype(f32)

    att_rep = jnp.repeat(att_w.astype(f32), H, axis=1)

    cnts = _sc_count_call([d_if, d_ff, d_of, d_fi, d_fo], ones_hbm, zeros_hbm)
    c_if, c_ff, c_of, c_fi, c_fo = cnts

    xfm, t_ff, t_fi, t_fo = _init_f_call(
        xf, m_pad, jnp.stack([Wl0[2], Wl0[1], Wl0[4]]))
    t_if = _init_io_call(xi, Wl0[0])
    t_of = _init_io_call(xo, Wl0[3])

    x_f, x_i, x_o = xfm, xi, xo
    idx_pairs = [(s_if, d_if), (s_ff, d_ff), (s_of, d_of),
                 (s_fi, d_fi), (s_fo, d_fo)]

    for l in range(5):
        if l == 0:
            WL, WR, BL = Wl0, Wr0, bl0
        else:
            WL, WR, BL = Wl[l - 1], Wr[l - 1], bl[l - 1]

        p_if, p_ff, p_of, p_fi, p_fo = _sc_agg_call(
            [t_if, t_ff, t_of, t_fi, t_fo], idx_pairs, zeros_hbm)

        wl3 = None if l == 0 else jnp.stack([WL[0], WL[2], WL[3]])
        wr3 = jnp.stack([WR[0], WR[2], WR[3]]).astype(f32)
        b_sum = (BL[0] + BL[2] + BL[3]).astype(f32)[None, :]

        x_f = _update_f_call(
            (p_if, p_ff, p_of), (c_if, c_ff, c_of), x_f, m_pad,
            wl3, wr3, b_sum, ln_w2, ln_b2)
        x_i = _update_io_call(
            p_fi, c_fi, x_i, None if l == 0 else WL[1],
            WR[1].astype(f32), BL[1][None, :].astype(f32), ln_w2, ln_b2)
        x_o = _update_io_call(
            p_fo, c_fo, x_o, None if l == 0 else WL[4],
            WR[4].astype(f32), BL[4][None, :].astype(f32), ln_w2, ln_b2)

        t_if, t_ff, t_of, t_fi, t_fo = x_i, x_f, x_o, x_f, x_f

    out = _pool_call(x_f, m_pad, batch_row, att_rep,
                     lin_w.astype(f32), lin_b.astype(f32)[None, :])
    return out
